# baseline (speedup 1.0000x reference)
import functools

import jax
import jax.numpy as jnp
from jax import lax
from jax.experimental import pallas as pl
from jax.experimental.pallas import tpu as pltpu
from jax.experimental.pallas import tpu_sc as plsc

_TOPIC_K = 128
_KW = _TOPIC_K // 2
_BATCH = 8192
_ONE_HOP = 32
_TWO_HOP = 64

_NC = 2
_NS = 16
_NW = _NC * _NS
_EPW = _BATCH // _NW
_L = 16
_NV = _TOPIC_K // _L
_NQ = _KW // _L
_NBUF = 5


def _unpack_word(w):
    lo = lax.bitcast_convert_type(w << 16, jnp.float32)
    hi = lax.bitcast_convert_type(w & jnp.int32(-65536), jnp.float32)
    return lo, hi


def _row_sum(rows_ref, slot, nrows, unroll):
    def body(i, acc):
        r = pl.multiple_of(i * unroll, unroll)
        new = list(acc)
        for u in range(unroll):
            for q in range(_NQ):
                a, b = _unpack_word(rows_ref[slot, r + u, pl.ds(q * _L, _L)])
                new[q] = new[q] + a
                new[_NQ + q] = new[_NQ + q] + b
        return tuple(new)

    zero = tuple(jnp.zeros((_L,), jnp.float32) for _ in range(_NV))
    return lax.fori_loop(0, nrows // unroll, body, zero)


def _sc_body(doc_tab, word_tab, didx_hbm, widx_hbm, v_hbm, out_hbm,
             didx_v, widx_v, vidx_v, rows_s, rows_d, rows_w, out_v,
             sem_idx, *sems):
    wid = lax.axis_index("s") * _NC + lax.axis_index("c")
    base = wid * _EPW

    pltpu.async_copy(
        didx_hbm.at[pl.ds(base * _TWO_HOP, _EPW * _TWO_HOP)], didx_v,
        sem_idx).wait()
    pltpu.async_copy(
        widx_hbm.at[pl.ds(base * _ONE_HOP, _EPW * _ONE_HOP)], widx_v,
        sem_idx).wait()
    pltpu.async_copy(v_hbm.at[pl.ds(base, _EPW)], vidx_v, sem_idx).wait()

    h = _EPW // 2
    pltpu.async_copy(
        doc_tab.at[vidx_v.at[pl.ds(0, h)]], rows_s.at[pl.ds(0, h)], sem_idx)
    pltpu.async_copy(
        doc_tab.at[vidx_v.at[pl.ds(h, h)]], rows_s.at[pl.ds(h, h)], sem_idx)
    pltpu.make_async_copy(
        doc_tab.at[vidx_v.at[pl.ds(0, h)]], rows_s.at[pl.ds(0, h)],
        sem_idx).wait()
    pltpu.make_async_copy(
        doc_tab.at[vidx_v.at[pl.ds(h, h)]], rows_s.at[pl.ds(h, h)],
        sem_idx).wait()

    sem_d = sems[:_NBUF]
    sem_w = sems[_NBUF:]

    def didx_at(e):
        return didx_v.at[pl.ds(pl.multiple_of(e * _TWO_HOP, 8), _TWO_HOP)]

    def widx_at(e):
        return widx_v.at[pl.ds(pl.multiple_of(e * _ONE_HOP, 8), _ONE_HOP)]

    def issue(e, slot):
        pltpu.async_copy(doc_tab.at[didx_at(e)], rows_d.at[slot], sem_d[slot])
        pltpu.async_copy(word_tab.at[widx_at(e)], rows_w.at[slot], sem_w[slot])

    def wait(slot):
        pltpu.make_async_copy(
            doc_tab.at[didx_at(0)], rows_d.at[slot], sem_d[slot]).wait()
        pltpu.make_async_copy(
            word_tab.at[widx_at(0)], rows_w.at[slot], sem_w[slot]).wait()

    lanes = lax.iota(jnp.int32, _L)

    def compute(e, slot):
        th = _row_sum(rows_d, slot, _TWO_HOP, 4)
        oh = _row_sum(rows_w, slot, _ONE_HOP, 4)
        acc = [None] * _NV
        for q in range(_NQ):
            sa, sb = _unpack_word(rows_s[e, pl.ds(q * _L, _L)])
            acc[q] = (sa + th[q] * (1.0 / _TWO_HOP)
                      + oh[q] * (1.0 / _ONE_HOP))
            acc[_NQ + q] = (sb + th[_NQ + q] * (1.0 / _TWO_HOP)
                            + oh[_NQ + q] * (1.0 / _ONE_HOP))

        def shuffle(x, idx):
            return x.at[idx].get(mode="promise_in_bounds")

        m = acc[0]
        for k in range(1, _NV):
            m = jnp.maximum(m, acc[k])
        for st in (8, 4, 2, 1):
            m = jnp.maximum(m, shuffle(m, lanes ^ st))
        ex = [jnp.exp(a - m) for a in acc]
        s = ex[0]
        for k in range(1, _NV):
            s = s + ex[k]
        for st in (8, 4, 2, 1):
            s = s + shuffle(s, lanes ^ st)
        r = 1.0 / s
        for k in range(_NV):
            out_v[e, pl.ds(k * _L, _L)] = ex[k] * r

    for p in range(_NBUF - 1):
        issue(p, p)

    def group(i, carry):
        e0 = i * _NBUF
        for par in range(_NBUF):
            e = e0 + par

            @pl.when(e + _NBUF - 1 < _EPW)
            def _():
                issue(e + _NBUF - 1, (par + _NBUF - 1) % _NBUF)

            wait(par)
            compute(e, par)
        return carry

    lax.fori_loop(0, _EPW // _NBUF, group, 0)

    for e in range((_EPW // _NBUF) * _NBUF, _EPW):
        wait(e % _NBUF)
        compute(e, e % _NBUF)

    pltpu.async_copy(out_v, out_hbm.at[pl.ds(base, _EPW)], sem_idx).wait()


@functools.partial(
    pl.kernel,
    out_type=jax.ShapeDtypeStruct((_BATCH, _TOPIC_K), jnp.float32),
    mesh=plsc.VectorSubcoreMesh(core_axis_name="c", subcore_axis_name="s"),
    compiler_params=pltpu.CompilerParams(use_tc_tiling_on_sc=False),
    scratch_types=[
        pltpu.VMEM((_EPW * _TWO_HOP,), jnp.int32),
        pltpu.VMEM((_EPW * _ONE_HOP,), jnp.int32),
        pltpu.VMEM((_EPW,), jnp.int32),
        pltpu.VMEM((_EPW, _KW), jnp.int32),
        pltpu.VMEM((_NBUF, _TWO_HOP, _KW), jnp.int32),
        pltpu.VMEM((_NBUF, _ONE_HOP, _KW), jnp.int32),
        pltpu.VMEM((_EPW, _TOPIC_K), jnp.float32),
    ] + [pltpu.SemaphoreType.DMA] * (1 + 2 * _NBUF),
)
def _topic_sc_kernel(doc_tab, word_tab, didx_hbm, widx_hbm, v_hbm, out_hbm,
                     *rest):
    _sc_body(doc_tab, word_tab, didx_hbm, widx_hbm, v_hbm, out_hbm, *rest)


def _pack_table(tab):
    xu = lax.bitcast_convert_type(tab, jnp.uint32)
    xb = (xu + ((xu >> 16) & 1) + 0x7FFF) >> 16
    w = xb | (jnp.roll(xb, -_KW, axis=1) << 16)
    return lax.bitcast_convert_type(w[:, :_KW], jnp.int32)


def kernel(v, one_hop_list, two_hop_list, doc_topic_dist, word_topic_dist):
    return _topic_sc_kernel(
        _pack_table(doc_topic_dist), _pack_table(word_topic_dist),
        two_hop_list.astype(jnp.int32).reshape(-1),
        one_hop_list.astype(jnp.int32).reshape(-1),
        v.astype(jnp.int32))


# speedup vs baseline: 2.1642x; 2.1642x over previous
import functools

import jax
import jax.numpy as jnp
from jax import lax
from jax.experimental import pallas as pl
from jax.experimental.pallas import tpu as pltpu
from jax.experimental.pallas import tpu_sc as plsc

_TOPIC_K = 128
_KW = _TOPIC_K // 2
_BATCH = 8192
_ONE_HOP = 32
_TWO_HOP = 64

_NC = 2
_NS = 16
_NW = _NC * _NS
_EPW = _BATCH // _NW
_L = 16
_NV = _TOPIC_K // _L
_NQ = _KW // _L
_NBUF = 3


def _row_sum(rows_ref, slot, nrows, unroll):
    def body(i, acc):
        r = pl.multiple_of(i * unroll, unroll)
        new = list(acc)
        for u in range(unroll):
            for k in range(_NV):
                new[k] = new[k] + rows_ref[slot, r + u, pl.ds(k * _L, _L)]
        return tuple(new)

    zero = tuple(jnp.zeros((_L,), jnp.float32) for _ in range(_NV))
    return lax.fori_loop(0, nrows // unroll, body, zero)


def _sc_body(doc_tab, word_tab, didx_hbm, widx_hbm, v_hbm, out_hbm,
             didx_v, widx_v, vidx_v, rows_s, rows_d, rows_w, out_v,
             sem_idx, *sems):
    wid = lax.axis_index("s") * _NC + lax.axis_index("c")
    base = wid * _EPW

    pltpu.async_copy(
        didx_hbm.at[pl.ds(base * _TWO_HOP, _EPW * _TWO_HOP)], didx_v,
        sem_idx).wait()
    pltpu.async_copy(
        widx_hbm.at[pl.ds(base * _ONE_HOP, _EPW * _ONE_HOP)], widx_v,
        sem_idx).wait()
    pltpu.async_copy(v_hbm.at[pl.ds(base, _EPW)], vidx_v, sem_idx).wait()

    h = _EPW // 2
    pltpu.async_copy(
        doc_tab.at[vidx_v.at[pl.ds(0, h)]], rows_s.at[pl.ds(0, h)], sem_idx)
    pltpu.async_copy(
        doc_tab.at[vidx_v.at[pl.ds(h, h)]], rows_s.at[pl.ds(h, h)], sem_idx)
    pltpu.make_async_copy(
        doc_tab.at[vidx_v.at[pl.ds(0, h)]], rows_s.at[pl.ds(0, h)],
        sem_idx).wait()
    pltpu.make_async_copy(
        doc_tab.at[vidx_v.at[pl.ds(h, h)]], rows_s.at[pl.ds(h, h)],
        sem_idx).wait()

    sem_d = sems[:_NBUF]
    sem_w = sems[_NBUF:]

    def didx_at(e):
        return didx_v.at[pl.ds(pl.multiple_of(e * _TWO_HOP, 8), _TWO_HOP)]

    def widx_at(e):
        return widx_v.at[pl.ds(pl.multiple_of(e * _ONE_HOP, 8), _ONE_HOP)]

    def issue(e, slot):
        pltpu.async_copy(doc_tab.at[didx_at(e)], rows_d.at[slot], sem_d[slot])
        pltpu.async_copy(word_tab.at[widx_at(e)], rows_w.at[slot], sem_w[slot])

    def wait(slot):
        pltpu.make_async_copy(
            doc_tab.at[didx_at(0)], rows_d.at[slot], sem_d[slot]).wait()
        pltpu.make_async_copy(
            word_tab.at[widx_at(0)], rows_w.at[slot], sem_w[slot]).wait()

    lanes = lax.iota(jnp.int32, _L)

    def compute(e, slot):
        th = _row_sum(rows_d, slot, _TWO_HOP, 4)
        oh = _row_sum(rows_w, slot, _ONE_HOP, 4)
        acc = [rows_s[e, pl.ds(k * _L, _L)]
               + th[k] * (1.0 / _TWO_HOP)
               + oh[k] * (1.0 / _ONE_HOP)
               for k in range(_NV)]

        def shuffle(x, idx):
            return x.at[idx].get(mode="promise_in_bounds")

        m = acc[0]
        for k in range(1, _NV):
            m = jnp.maximum(m, acc[k])
        for st in (8, 4, 2, 1):
            m = jnp.maximum(m, shuffle(m, lanes ^ st))
        ex = [jnp.exp(a - m) for a in acc]
        s = ex[0]
        for k in range(1, _NV):
            s = s + ex[k]
        for st in (8, 4, 2, 1):
            s = s + shuffle(s, lanes ^ st)
        r = 1.0 / s
        for k in range(_NV):
            out_v[e, pl.ds(k * _L, _L)] = ex[k] * r

    for p in range(_NBUF - 1):
        issue(p, p)

    def group(i, carry):
        e0 = i * _NBUF
        for par in range(_NBUF):
            e = e0 + par

            @pl.when(e + _NBUF - 1 < _EPW)
            def _():
                issue(e + _NBUF - 1, (par + _NBUF - 1) % _NBUF)

            wait(par)
            compute(e, par)
        return carry

    lax.fori_loop(0, _EPW // _NBUF, group, 0)

    for e in range((_EPW // _NBUF) * _NBUF, _EPW):
        wait(e % _NBUF)
        compute(e, e % _NBUF)

    pltpu.async_copy(out_v, out_hbm.at[pl.ds(base, _EPW)], sem_idx).wait()


@functools.partial(
    pl.kernel,
    out_type=jax.ShapeDtypeStruct((_BATCH, _TOPIC_K), jnp.float32),
    mesh=plsc.VectorSubcoreMesh(core_axis_name="c", subcore_axis_name="s"),
    compiler_params=pltpu.CompilerParams(use_tc_tiling_on_sc=False),
    scratch_types=[
        pltpu.VMEM((_EPW * _TWO_HOP,), jnp.int32),
        pltpu.VMEM((_EPW * _ONE_HOP,), jnp.int32),
        pltpu.VMEM((_EPW,), jnp.int32),
        pltpu.VMEM((_EPW, _TOPIC_K), jnp.float32),
        pltpu.VMEM((_NBUF, _TWO_HOP, _TOPIC_K), jnp.float32),
        pltpu.VMEM((_NBUF, _ONE_HOP, _TOPIC_K), jnp.float32),
        pltpu.VMEM((_EPW, _TOPIC_K), jnp.float32),
    ] + [pltpu.SemaphoreType.DMA] * (1 + 2 * _NBUF),
)
def _topic_sc_kernel(doc_tab, word_tab, didx_hbm, widx_hbm, v_hbm, out_hbm,
                     *rest):
    _sc_body(doc_tab, word_tab, didx_hbm, widx_hbm, v_hbm, out_hbm, *rest)


def kernel(v, one_hop_list, two_hop_list, doc_topic_dist, word_topic_dist):
    return _topic_sc_kernel(
        doc_topic_dist, word_topic_dist,
        two_hop_list.astype(jnp.int32).reshape(-1),
        one_hop_list.astype(jnp.int32).reshape(-1),
        v.astype(jnp.int32))


# speedup vs baseline: 2.3535x; 1.0875x over previous
import functools

import jax
import jax.numpy as jnp
from jax import lax
from jax.experimental import pallas as pl
from jax.experimental.pallas import tpu as pltpu
from jax.experimental.pallas import tpu_sc as plsc

_TOPIC_K = 128
_BATCH = 8192
_ONE_HOP = 32
_TWO_HOP = 64
_DOC_ROWS = 1 + _TWO_HOP
_DSTRIDE = 72

_NC = 2
_NS = 16
_NW = _NC * _NS
_EPW = _BATCH // _NW
_L = 16
_NV = _TOPIC_K // _L
_NBUF = 5


def _row_sum(rows_ref, slot, start, nrows, unroll):
    def body(i, acc):
        r = start + i * unroll
        new = list(acc)
        for u in range(unroll):
            for k in range(_NV):
                new[k] = new[k] + rows_ref[slot, r + u, pl.ds(k * _L, _L)]
        return tuple(new)

    zero = tuple(jnp.zeros((_L,), jnp.float32) for _ in range(_NV))
    return lax.fori_loop(0, nrows // unroll, body, zero)


def _sc_body(doc_tab, word_tab, didx_hbm, widx_hbm, out_hbm,
             didx_v, widx_v, rows_d, rows_w, out_v,
             sem_idx, *sems):
    wid = lax.axis_index("s") * _NC + lax.axis_index("c")
    base = wid * _EPW

    pltpu.async_copy(
        didx_hbm.at[pl.ds(base * _DSTRIDE, _EPW * _DSTRIDE)], didx_v,
        sem_idx).wait()
    pltpu.async_copy(
        widx_hbm.at[pl.ds(base * _ONE_HOP, _EPW * _ONE_HOP)], widx_v,
        sem_idx).wait()

    sem_d = sems[:_NBUF]
    sem_w = sems[_NBUF:]

    def didx_at(e):
        return didx_v.at[pl.ds(pl.multiple_of(e * _DSTRIDE, 8), _DOC_ROWS)]

    def widx_at(e):
        return widx_v.at[pl.ds(pl.multiple_of(e * _ONE_HOP, 8), _ONE_HOP)]

    def issue(e, slot):
        pltpu.async_copy(doc_tab.at[didx_at(e)], rows_d.at[slot], sem_d[slot])
        pltpu.async_copy(word_tab.at[widx_at(e)], rows_w.at[slot], sem_w[slot])

    def wait(slot):
        pltpu.make_async_copy(
            doc_tab.at[didx_at(0)], rows_d.at[slot], sem_d[slot]).wait()
        pltpu.make_async_copy(
            word_tab.at[widx_at(0)], rows_w.at[slot], sem_w[slot]).wait()

    lanes = lax.iota(jnp.int32, _L)

    def compute(e, slot):
        th = _row_sum(rows_d, slot, 1, _TWO_HOP, 4)
        oh = _row_sum(rows_w, slot, 0, _ONE_HOP, 4)
        acc = [
            rows_d[slot, 0, pl.ds(k * _L, _L)]
            + th[k] * (1.0 / _TWO_HOP)
            + oh[k] * (1.0 / _ONE_HOP)
            for k in range(_NV)
        ]

        def shuffle(x, idx):
            return x.at[idx].get(mode="promise_in_bounds")

        m = acc[0]
        for k in range(1, _NV):
            m = jnp.maximum(m, acc[k])
        for st in (8, 4, 2, 1):
            m = jnp.maximum(m, shuffle(m, lanes ^ st))
        ex = [jnp.exp(a - m) for a in acc]
        s = ex[0]
        for k in range(1, _NV):
            s = s + ex[k]
        for st in (8, 4, 2, 1):
            s = s + shuffle(s, lanes ^ st)
        r = 1.0 / s
        for k in range(_NV):
            out_v[e, pl.ds(k * _L, _L)] = ex[k] * r

    for p in range(_NBUF - 1):
        issue(p, p)

    def group(i, carry):
        e0 = i * _NBUF
        for par in range(_NBUF):
            e = e0 + par

            @pl.when(e + _NBUF - 1 < _EPW)
            def _():
                issue(e + _NBUF - 1, (par + _NBUF - 1) % _NBUF)

            wait(par)
            compute(e, par)
        return carry

    lax.fori_loop(0, _EPW // _NBUF, group, 0)

    for e in range((_EPW // _NBUF) * _NBUF, _EPW):
        wait(e % _NBUF)
        compute(e, e % _NBUF)

    pltpu.async_copy(out_v, out_hbm.at[pl.ds(base, _EPW)], sem_idx).wait()


@functools.partial(
    pl.kernel,
    out_type=jax.ShapeDtypeStruct((_BATCH, _TOPIC_K), jnp.float32),
    mesh=plsc.VectorSubcoreMesh(core_axis_name="c", subcore_axis_name="s"),
    scratch_types=[
        pltpu.VMEM((_EPW * _DSTRIDE,), jnp.int32),
        pltpu.VMEM((_EPW * _ONE_HOP,), jnp.int32),
        pltpu.VMEM((_NBUF, _DOC_ROWS, _TOPIC_K), jnp.float32),
        pltpu.VMEM((_NBUF, _ONE_HOP, _TOPIC_K), jnp.float32),
        pltpu.VMEM((_EPW, _TOPIC_K), jnp.float32),
    ] + [pltpu.SemaphoreType.DMA] * (1 + 2 * _NBUF),
)
def _topic_sc_kernel(doc_tab, word_tab, didx_hbm, widx_hbm, out_hbm, *rest):
    _sc_body(doc_tab, word_tab, didx_hbm, widx_hbm, out_hbm, *rest)


def kernel(v, one_hop_list, two_hop_list, doc_topic_dist, word_topic_dist):
    didx = jnp.concatenate(
        [v.astype(jnp.int32)[:, None], two_hop_list.astype(jnp.int32)], axis=1)
    didx = jnp.pad(didx, ((0, 0), (0, _DSTRIDE - _DOC_ROWS)))
    widx = one_hop_list.astype(jnp.int32)
    return _topic_sc_kernel(
        doc_topic_dist, word_topic_dist, didx.reshape(-1), widx.reshape(-1))


# speedup vs baseline: 2.3775x; 1.0102x over previous
import functools

import jax
import jax.numpy as jnp
from jax import lax
from jax.experimental import pallas as pl
from jax.experimental.pallas import tpu as pltpu
from jax.experimental.pallas import tpu_sc as plsc

_TOPIC_K = 128
_BATCH = 8192
_ONE_HOP = 32
_TWO_HOP = 64
_DOC_ROWS = 1 + _TWO_HOP
_DSTRIDE = 72

_NC = 2
_NS = 16
_NW = _NC * _NS
_EPW = _BATCH // _NW
_L = 16
_NV = _TOPIC_K // _L
_NBUF = 5


def _row_sum(rows_ref, slot, start, nrows, unroll):
    def body(i, acc):
        r = start + i * unroll
        new = list(acc)
        for u in range(unroll):
            for k in range(_NV):
                new[k] = new[k] + rows_ref[slot, r + u, pl.ds(k * _L, _L)]
        return tuple(new)

    zero = tuple(jnp.zeros((_L,), jnp.float32) for _ in range(_NV))
    return lax.fori_loop(0, nrows // unroll, body, zero)


def _sc_body(doc_tab, word_tab, didx_hbm, widx_hbm, out_hbm,
             didx_v, widx_v, rows_d, rows_w, out_v,
             sem_idx, *sems):
    wid = lax.axis_index("s") * _NC + lax.axis_index("c")
    base = wid * _EPW

    c_d = pltpu.async_copy(
        didx_hbm.at[pl.ds(base * _DSTRIDE, _EPW * _DSTRIDE)], didx_v, sem_idx)
    c_w = pltpu.async_copy(
        widx_hbm.at[pl.ds(base * _ONE_HOP, _EPW * _ONE_HOP)], widx_v, sem_idx)
    c_d.wait()
    c_w.wait()

    sem_d = sems[:_NBUF]
    sem_w = sems[_NBUF:]

    def didx_at(e):
        return didx_v.at[pl.ds(pl.multiple_of(e * _DSTRIDE, 8), _DOC_ROWS)]

    def widx_at(e):
        return widx_v.at[pl.ds(pl.multiple_of(e * _ONE_HOP, 8), _ONE_HOP)]

    def issue(e, slot):
        pltpu.async_copy(doc_tab.at[didx_at(e)], rows_d.at[slot], sem_d[slot])
        pltpu.async_copy(word_tab.at[widx_at(e)], rows_w.at[slot], sem_w[slot])

    def wait(slot):
        pltpu.make_async_copy(
            doc_tab.at[didx_at(0)], rows_d.at[slot], sem_d[slot]).wait()
        pltpu.make_async_copy(
            word_tab.at[widx_at(0)], rows_w.at[slot], sem_w[slot]).wait()

    lanes = lax.iota(jnp.int32, _L)

    def compute(e, slot):
        th = _row_sum(rows_d, slot, 1, _TWO_HOP, 4)
        oh = _row_sum(rows_w, slot, 0, _ONE_HOP, 4)
        acc = [
            rows_d[slot, 0, pl.ds(k * _L, _L)]
            + th[k] * (1.0 / _TWO_HOP)
            + oh[k] * (1.0 / _ONE_HOP)
            for k in range(_NV)
        ]

        def shuffle(x, idx):
            return x.at[idx].get(mode="promise_in_bounds")

        m = acc[0]
        for k in range(1, _NV):
            m = jnp.maximum(m, acc[k])
        for st in (8, 4, 2, 1):
            m = jnp.maximum(m, shuffle(m, lanes ^ st))
        ex = [jnp.exp(a - m) for a in acc]
        s = ex[0]
        for k in range(1, _NV):
            s = s + ex[k]
        for st in (8, 4, 2, 1):
            s = s + shuffle(s, lanes ^ st)
        r = 1.0 / s
        for k in range(_NV):
            out_v[e, pl.ds(k * _L, _L)] = ex[k] * r

    for p in range(_NBUF - 1):
        issue(p, p)

    def group(i, carry):
        e0 = i * _NBUF
        for par in range(_NBUF):
            e = e0 + par

            @pl.when(e + _NBUF - 1 < _EPW)
            def _():
                issue(e + _NBUF - 1, (par + _NBUF - 1) % _NBUF)

            wait(par)
            compute(e, par)
        return carry

    lax.fori_loop(0, _EPW // _NBUF, group, 0)

    for e in range((_EPW // _NBUF) * _NBUF, _EPW):
        wait(e % _NBUF)
        compute(e, e % _NBUF)

    pltpu.async_copy(out_v, out_hbm.at[pl.ds(base, _EPW)], sem_idx).wait()


@functools.partial(
    pl.kernel,
    out_type=jax.ShapeDtypeStruct((_BATCH, _TOPIC_K), jnp.float32),
    mesh=plsc.VectorSubcoreMesh(core_axis_name="c", subcore_axis_name="s"),
    scratch_types=[
        pltpu.VMEM((_EPW * _DSTRIDE,), jnp.int32),
        pltpu.VMEM((_EPW * _ONE_HOP,), jnp.int32),
        pltpu.VMEM((_NBUF, _DOC_ROWS, _TOPIC_K), jnp.float32),
        pltpu.VMEM((_NBUF, _ONE_HOP, _TOPIC_K), jnp.float32),
        pltpu.VMEM((_EPW, _TOPIC_K), jnp.float32),
    ] + [pltpu.SemaphoreType.DMA] * (1 + 2 * _NBUF),
)
def _topic_sc_kernel(doc_tab, word_tab, didx_hbm, widx_hbm, out_hbm, *rest):
    _sc_body(doc_tab, word_tab, didx_hbm, widx_hbm, out_hbm, *rest)


def kernel(v, one_hop_list, two_hop_list, doc_topic_dist, word_topic_dist):
    didx = jnp.concatenate(
        [v.astype(jnp.int32)[:, None], two_hop_list.astype(jnp.int32)], axis=1)
    didx = jnp.pad(didx, ((0, 0), (0, _DSTRIDE - _DOC_ROWS)))
    widx = one_hop_list.astype(jnp.int32)
    return _topic_sc_kernel(
        doc_topic_dist, word_topic_dist, didx.reshape(-1), widx.reshape(-1))
